# speedup vs baseline: 4.1439x; 4.1439x over previous
import functools

import jax
import jax.numpy as jnp
from jax import lax
from jax.experimental import pallas as pl
from jax.experimental.pallas import tpu as pltpu
from jax.experimental.pallas import tpu_sc as plsc

N_ROWS = 100000
D = 128
N_SEG = 1024
NC = 2
NS = 16
HALF = N_ROWS // NC
UNIT = 128
FULL_UNITS = HALF // UNIT
TAIL = HALF - FULL_UNITS * UNIT
SEG_PER_SUB = N_SEG // NS
MAX_K = (FULL_UNITS + NS - 1) // NS


def _sc_partials(X, I, Z):
    mesh = plsc.VectorSubcoreMesh(core_axis_name="c", subcore_axis_name="s")

    @functools.partial(
        pl.kernel,
        out_type=jax.ShapeDtypeStruct((NC * N_SEG, D), jnp.float32),
        mesh=mesh,
        scratch_types=[
            pltpu.VMEM((UNIT,), jnp.int32),
            pltpu.VMEM((UNIT, D), jnp.float32),
            pltpu.VMEM((TAIL,), jnp.int32),
            pltpu.VMEM((TAIL, D), jnp.float32),
            pltpu.VMEM_SHARED((N_SEG, D), jnp.float32),
        ],
    )
    def body(x_hbm, i_hbm, z_hbm, out_hbm, idx_v, rows_v, idxt_v, rowst_v, acc_sh):
        c = lax.axis_index("c")
        s = lax.axis_index("s")
        pltpu.sync_copy(
            z_hbm.at[pl.ds(s * SEG_PER_SUB, SEG_PER_SUB)],
            acc_sh.at[pl.ds(s * SEG_PER_SUB, SEG_PER_SUB)],
        )
        plsc.subcore_barrier()

        base = c * HALF

        def step(k, carry):
            u = s + k * NS

            @pl.when(u < FULL_UNITS)
            def _():
                r0 = base + u * UNIT
                pltpu.sync_copy(i_hbm.at[pl.ds(r0, UNIT)], idx_v)
                pltpu.sync_copy(x_hbm.at[pl.ds(r0, UNIT)], rows_v)
                pltpu.sync_copy(rows_v, acc_sh.at[idx_v], add=True)

            return carry

        lax.fori_loop(0, MAX_K, step, 0)

        @pl.when(s == NS - 1)
        def _():
            r0 = base + FULL_UNITS * UNIT
            pltpu.sync_copy(i_hbm.at[pl.ds(r0, TAIL)], idxt_v)
            pltpu.sync_copy(x_hbm.at[pl.ds(r0, TAIL)], rowst_v)
            pltpu.sync_copy(rowst_v, acc_sh.at[idxt_v], add=True)

        plsc.subcore_barrier()
        pltpu.sync_copy(
            acc_sh.at[pl.ds(s * SEG_PER_SUB, SEG_PER_SUB)],
            out_hbm.at[pl.ds(c * N_SEG + s * SEG_PER_SUB, SEG_PER_SUB)],
        )

    return body(X, I, Z)


def _combine(p):
    def body(p_ref, o_ref):
        o_ref[...] = p_ref[0] + p_ref[1]

    blk = N_SEG // 8
    return pl.pallas_call(
        body,
        out_shape=jax.ShapeDtypeStruct((N_SEG, D), jnp.float32),
        grid=(8,),
        in_specs=[pl.BlockSpec((2, blk, D), lambda i: (0, i, 0))],
        out_specs=pl.BlockSpec((blk, D), lambda i: (i, 0)),
    )(p)


def kernel(X, I):
    if I.ndim == 2:
        I = I[:, 0]
    I = I.astype(jnp.int32)
    Z = jnp.zeros((N_SEG, D), jnp.float32)
    partials = _sc_partials(X, I, Z)
    return _combine(partials.reshape(NC, N_SEG, D))


# speedup vs baseline: 6.3324x; 1.5281x over previous
import functools

import jax
import jax.numpy as jnp
from jax import lax
from jax.experimental import pallas as pl
from jax.experimental.pallas import tpu as pltpu
from jax.experimental.pallas import tpu_sc as plsc

N_ROWS = 100000
D = 128
N_SEG = 1024
NC = 2
NS = 16
HALF = N_ROWS // NC
UNIT = 128
FULL_UNITS = HALF // UNIT
TAIL = HALF - FULL_UNITS * UNIT
SEG_PER_SUB = N_SEG // NS
MAX_K = (FULL_UNITS + NS - 1) // NS


def _sc_partials(X, I, Z):
    mesh = plsc.VectorSubcoreMesh(core_axis_name="c", subcore_axis_name="s")

    @functools.partial(
        pl.kernel,
        out_type=jax.ShapeDtypeStruct((NC * N_SEG, D), jnp.float32),
        mesh=mesh,
        scratch_types=[
            pltpu.VMEM((2, UNIT), jnp.int32),
            pltpu.VMEM((2, UNIT, D), jnp.float32),
            pltpu.VMEM((TAIL,), jnp.int32),
            pltpu.VMEM((TAIL, D), jnp.float32),
            pltpu.VMEM_SHARED((N_SEG, D), jnp.float32),
            pltpu.SemaphoreType.DMA,
            pltpu.SemaphoreType.DMA,
        ],
    )
    def body(x_hbm, i_hbm, z_hbm, out_hbm, idx_v, rows_v, idxt_v, rowst_v,
             acc_sh, sem0, sem1):
        c = lax.axis_index("c")
        s = lax.axis_index("s")
        pltpu.sync_copy(
            z_hbm.at[pl.ds(s * SEG_PER_SUB, SEG_PER_SUB)],
            acc_sh.at[pl.ds(s * SEG_PER_SUB, SEG_PER_SUB)],
        )
        plsc.subcore_barrier()

        base = c * HALF
        sems = (sem0, sem1)

        def fetch(k, bi):
            u = s + k * NS

            @pl.when(u < FULL_UNITS)
            def _():
                r0 = base + u * UNIT
                pltpu.async_copy(i_hbm.at[pl.ds(r0, UNIT)], idx_v.at[bi], sems[bi])
                pltpu.async_copy(x_hbm.at[pl.ds(r0, UNIT)], rows_v.at[bi], sems[bi])

        def flush(k, bi):
            u = s + k * NS

            @pl.when(u < FULL_UNITS)
            def _():
                r0 = base + u * UNIT
                pltpu.make_async_copy(
                    i_hbm.at[pl.ds(r0, UNIT)], idx_v.at[bi], sems[bi]).wait()
                pltpu.make_async_copy(
                    x_hbm.at[pl.ds(r0, UNIT)], rows_v.at[bi], sems[bi]).wait()
                pltpu.sync_copy(rows_v.at[bi], acc_sh.at[idx_v.at[bi]], add=True)

        fetch(0, 0)

        def step(kk, carry):
            k0 = 2 * kk
            fetch(k0 + 1, 1)
            flush(k0, 0)
            fetch(k0 + 2, 0)
            flush(k0 + 1, 1)
            return carry

        lax.fori_loop(0, (MAX_K + 1) // 2, step, 0)

        @pl.when(s == NS - 1)
        def _():
            r0 = base + FULL_UNITS * UNIT
            pltpu.sync_copy(i_hbm.at[pl.ds(r0, TAIL)], idxt_v)
            pltpu.sync_copy(x_hbm.at[pl.ds(r0, TAIL)], rowst_v)
            pltpu.sync_copy(rowst_v, acc_sh.at[idxt_v], add=True)

        plsc.subcore_barrier()
        pltpu.sync_copy(
            acc_sh.at[pl.ds(s * SEG_PER_SUB, SEG_PER_SUB)],
            out_hbm.at[pl.ds(c * N_SEG + s * SEG_PER_SUB, SEG_PER_SUB)],
        )

    return body(X, I, Z)


def _combine(p):
    def body(p_ref, o_ref):
        o_ref[...] = p_ref[0] + p_ref[1]

    blk = N_SEG // 8
    return pl.pallas_call(
        body,
        out_shape=jax.ShapeDtypeStruct((N_SEG, D), jnp.float32),
        grid=(8,),
        in_specs=[pl.BlockSpec((2, blk, D), lambda i: (0, i, 0))],
        out_specs=pl.BlockSpec((blk, D), lambda i: (i, 0)),
    )(p)


def kernel(X, I):
    if I.ndim == 2:
        I = I[:, 0]
    I = I.astype(jnp.int32)
    Z = jnp.zeros((N_SEG, D), jnp.float32)
    partials = _sc_partials(X, I, Z)
    return _combine(partials.reshape(NC, N_SEG, D))
